# baseline (speedup 1.0000x reference)
import dataclasses
import functools

import jax
import jax.numpy as jnp
from jax import lax
from jax.experimental import pallas as pl
from jax.experimental.pallas import tpu as pltpu
from jax.experimental.pallas import tpu_sc as plsc

N = 10000
E = 320000
D = 128

NC = 2
NS = 16
NW = NC * NS
CHUNK = 125
NCHUNK = E // CHUNK
CPW = NCHUNK // NW
IQ = 16
RMAIN = 624
RTAIL = N - RMAIN * NS

_mesh = plsc.VectorSubcoreMesh(core_axis_name="c", subcore_axis_name="s")

_cp_no_layout = pltpu.CompilerParams()
if "needs_layout_passes" in pltpu.CompilerParams.__dataclass_fields__:
    _cp_no_layout = dataclasses.replace(_cp_no_layout,
                                        needs_layout_passes=False)


def _each_row_slice(s, fn):
    fn(s * RMAIN, RMAIN)

    @pl.when(s < RTAIL // 8)
    def _():
        fn(RMAIN * NS + s * 8, 8)


DROWS = 80
EPW = E // NW
NVEC = EPW // 16


def _sc_degree(dst1d):
    @functools.partial(
        pl.kernel,
        mesh=_mesh,
        out_type=jax.ShapeDtypeStruct((NC, DROWS, D), jnp.float32),
        compiler_params=_cp_no_layout,
        scratch_types=[
            pltpu.VMEM((EPW,), jnp.int32),
            pltpu.VMEM((8, DROWS, D), jnp.float32),
            pltpu.VMEM((1, DROWS), jnp.int32),
            pltpu.VMEM_SHARED((DROWS, D), jnp.float32),
        ],
    )
    def k(dst_hbm, out_hbm, idx_v, hist, rowid, accd):
        c = lax.axis_index("c")
        s = lax.axis_index("s")
        w = s * NC + c
        pltpu.sync_copy(dst_hbm.at[pl.ds(w * EPW, EPW)], idx_v)

        zeros16 = jnp.zeros((16,), jnp.float32)
        iota = lax.iota(jnp.int32, 16)

        @pl.loop(0, 8)
        def _(p):
            @pl.loop(0, DROWS)
            def _(r):
                @pl.loop(0, D // 16)
                def _(cc):
                    hist[p, r, pl.ds(cc * 16, 16)] = zeros16

        @pl.loop(0, DROWS // 16)
        def _(kk):
            rowid[0, pl.ds(kk * 16, 16)] = iota + kk * 16

        @pl.when(s == 0)
        def _():
            pltpu.sync_copy(hist.at[0], accd)

        plane = iota & 7
        mask_lo = iota < 8
        mask_hi = iota >= 8
        ones16 = jnp.ones((16,), jnp.float32)

        @pl.loop(0, NVEC)
        def _(v):
            d = idx_v[pl.ds(v * 16, 16)]
            r = lax.shift_right_logical(d, 7)
            col = d & 127
            plsc.addupdate_scatter(hist, [plane, r, col], ones16, mask=mask_lo)
            plsc.addupdate_scatter(hist, [plane, r, col], ones16, mask=mask_hi)

        plsc.subcore_barrier()

        @pl.loop(0, 8)
        def _(p):
            pltpu.sync_copy(hist.at[p], accd.at[rowid.at[0]], add=True)

        plsc.subcore_barrier()

        @pl.when(s < DROWS // 8)
        def _():
            pltpu.sync_copy(accd.at[pl.ds(s * 8, 8)],
                            out_hbm.at[c, pl.ds(s * 8, 8)])

    return k(dst1d)


def _sc_propagate(g, src2d, dst2d):
    @functools.partial(
        pl.kernel,
        mesh=_mesh,
        out_type=jax.ShapeDtypeStruct((NC, N, D), jnp.float32),
        scratch_types=[
            pltpu.VMEM((IQ, CHUNK), jnp.int32),
            pltpu.VMEM((IQ, CHUNK), jnp.int32),
            pltpu.VMEM((2, CHUNK, D), jnp.float32),
            pltpu.VMEM_SHARED((N, D), jnp.float32),
            pltpu.SemaphoreType.DMA((2,)),
            pltpu.SemaphoreType.DMA((2,)),
        ],
    )
    def k(g_hbm, src_hbm, dst_hbm, out_hbm, sidx, didx, rows, acc, gsem, ssem):
        c = lax.axis_index("c")
        s = lax.axis_index("s")
        w = s * NC + c
        pltpu.sync_copy(src_hbm.at[pl.ds(w * CPW, IQ)], sidx)
        pltpu.sync_copy(dst_hbm.at[pl.ds(w * CPW, IQ)], didx)
        _each_row_slice(s, lambda st, sz: pltpu.sync_copy(
            g_hbm.at[pl.ds(st, sz)], acc.at[pl.ds(st, sz)]))
        plsc.subcore_barrier()

        def g_start(j):
            pltpu.async_copy(g_hbm.at[sidx.at[j % IQ]], rows.at[j % 2],
                             gsem.at[j % 2])

        def g_wait(j):
            pltpu.make_async_copy(g_hbm.at[sidx.at[j % IQ]], rows.at[j % 2],
                                  gsem.at[j % 2]).wait()

        def s_start(j):
            pltpu.async_copy(rows.at[j % 2], acc.at[didx.at[j % IQ]],
                             ssem.at[j % 2], add=True)

        def s_wait(j):
            pltpu.make_async_copy(rows.at[j % 2], acc.at[didx.at[j % IQ]],
                                  ssem.at[j % 2]).wait()

        g_start(0)

        @pl.loop(0, CPW)
        def _(j):
            g_wait(j)

            @pl.when(j >= 1)
            def _():
                s_wait(j - 1)

            @pl.when(j < CPW - 1)
            def _():
                jj = j + 1
                @pl.when(jj % IQ == 0)
                def _():
                    off = pl.multiple_of(w * CPW + jj, 8)
                    pltpu.sync_copy(src_hbm.at[pl.ds(off, IQ)], sidx)

                g_start(jj)

            @pl.when(jnp.logical_and(j % IQ == 0, j > 0))
            def _():
                off = pl.multiple_of(w * CPW + j, 8)
                pltpu.sync_copy(dst_hbm.at[pl.ds(off, IQ)], didx)

            s_start(j)

        s_wait(CPW - 1)
        plsc.subcore_barrier()
        _each_row_slice(s, lambda st, sz: pltpu.sync_copy(
            acc.at[pl.ds(st, sz)], out_hbm.at[c, pl.ds(st, sz)]))

    return k(g, src2d, dst2d)


GB = 2000
NBLK = N // GB


def _row_spec(shape2):
    return pl.BlockSpec((GB, shape2), lambda i: (i, 0))


_full_w = pl.BlockSpec((D, D), lambda i: (0, 0))
_full_b = pl.BlockSpec((1, D), lambda i: (0, 0))
_zp_spec = pl.BlockSpec((NC, GB, D), lambda i: (0, i, 0))


def _tc_mm_scale0(x, W0, degp2):
    def body(x_ref, w_ref, degp_ref, g_ref, dinvb_ref):
        deg = degp_ref[0] + degp_ref[1] + 1.0
        dinvb = jnp.broadcast_to(lax.rsqrt(deg), (GB, D))
        dinvb_ref[...] = dinvb
        h0 = jnp.dot(x_ref[...], w_ref[...],
                     preferred_element_type=jnp.float32)
        g_ref[...] = dinvb * h0

    return pl.pallas_call(
        body,
        grid=(NBLK,),
        in_specs=[_row_spec(D), _full_w,
                  pl.BlockSpec((NC, GB, 1), lambda i: (0, i, 0))],
        out_specs=(_row_spec(D), _row_spec(D)),
        out_shape=(jax.ShapeDtypeStruct((N, D), jnp.float32),
                   jax.ShapeDtypeStruct((N, D), jnp.float32)))(x, W0, degp2)


def _tc_combine_stats(zp, g, dinvb, b0):
    def body(zp_ref, g_ref, dinvb_ref, b_ref, y_ref, st_ref, acc):
        i = pl.program_id(0)

        @pl.when(i == 0)
        def _():
            acc[...] = jnp.zeros((8, D), jnp.float32)

        z = zp_ref[0] + zp_ref[1] - g_ref[...]
        y = dinvb_ref[...] * z + b_ref[...]
        y_ref[...] = y
        acc[0:1, :] += jnp.sum(y, axis=0, keepdims=True)
        acc[1:2, :] += jnp.sum(y * y, axis=0, keepdims=True)

        @pl.when(i == NBLK - 1)
        def _():
            st_ref[...] = acc[...]

    return pl.pallas_call(
        body,
        grid=(NBLK,),
        in_specs=[_zp_spec, _row_spec(D), _row_spec(D), _full_b],
        out_specs=(_row_spec(D), pl.BlockSpec((8, D), lambda i: (0, 0))),
        out_shape=(jax.ShapeDtypeStruct((N, D), jnp.float32),
                   jax.ShapeDtypeStruct((8, D), jnp.float32)),
        scratch_shapes=[pltpu.VMEM((8, D), jnp.float32)])(zp, g, dinvb, b0)


def _tc_bn_relu_mm(y0, stats, gamma0, beta0, W1, dinvb):
    def body(y_ref, st_ref, gam_ref, bet_ref, w_ref, dinvb_ref, o_ref):
        mean = st_ref[0:1, :] * (1.0 / N)
        var = st_ref[1:2, :] * (1.0 / N) - mean * mean
        xn = (y_ref[...] - mean) * lax.rsqrt(var + 1e-5) * gam_ref[...] \
            + bet_ref[...]
        xr = jnp.maximum(xn, 0.0)
        h1 = jnp.dot(xr, w_ref[...], preferred_element_type=jnp.float32)
        o_ref[...] = dinvb_ref[...] * h1

    return pl.pallas_call(
        body,
        grid=(NBLK,),
        in_specs=[_row_spec(D), pl.BlockSpec((8, D), lambda i: (0, 0)),
                  _full_b, _full_b, _full_w, _row_spec(D)],
        out_specs=_row_spec(D),
        out_shape=jax.ShapeDtypeStruct((N, D), jnp.float32))(
            y0, stats, gamma0, beta0, W1, dinvb)


def _tc_combine_mm(zp, g, dinvb, b1, Wf):
    def body(zp_ref, g_ref, dinvb_ref, b_ref, w_ref, o_ref):
        dinvb = dinvb_ref[...]
        z = zp_ref[0] + zp_ref[1] - g_ref[...]
        y = dinvb * z + b_ref[...]
        h2 = jnp.dot(y, w_ref[...], preferred_element_type=jnp.float32)
        o_ref[...] = dinvb * h2

    return pl.pallas_call(
        body,
        grid=(NBLK,),
        in_specs=[_zp_spec, _row_spec(D), _row_spec(D), _full_b, _full_w],
        out_specs=_row_spec(D),
        out_shape=jax.ShapeDtypeStruct((N, D), jnp.float32))(
            zp, g, dinvb, b1, Wf)


def _tc_final(zp, g, dinvb, bf):
    def body(zp_ref, g_ref, dinvb_ref, b_ref, o_ref):
        z = zp_ref[0] + zp_ref[1] - g_ref[...]
        o = dinvb_ref[...] * z + b_ref[...]
        m = jnp.max(o, axis=1, keepdims=True)
        lse = jnp.log(jnp.sum(jnp.exp(o - m), axis=1, keepdims=True)) + m
        o_ref[...] = o - lse

    return pl.pallas_call(
        body,
        grid=(NBLK,),
        in_specs=[_zp_spec, _row_spec(D), _row_spec(D), _full_b],
        out_specs=_row_spec(D),
        out_shape=jax.ShapeDtypeStruct((N, D), jnp.float32))(zp, g, dinvb, bf)


def kernel(x, edge_index, W0, b0, gamma0, beta0, W1, b1, Wf, bf):
    ei = edge_index.astype(jnp.int32)
    src2d = ei[0].reshape(NCHUNK, CHUNK)
    dst2d = ei[1].reshape(NCHUNK, CHUNK)
    b0r = b0.reshape(1, D)
    gam = gamma0.reshape(1, D)
    bet = beta0.reshape(1, D)
    b1r = b1.reshape(1, D)
    bfr = bf.reshape(1, D)

    degp = _sc_degree(ei[1])
    degp2 = degp.reshape(NC, DROWS * D)[:, :N, None]
    g0, dinvb = _tc_mm_scale0(x, W0, degp2)

    zp0 = _sc_propagate(g0, src2d, dst2d)
    y0, stats = _tc_combine_stats(zp0, g0, dinvb, b0r)
    g1 = _tc_bn_relu_mm(y0, stats, gam, bet, W1, dinvb)

    zp1 = _sc_propagate(g1, src2d, dst2d)
    g2 = _tc_combine_mm(zp1, g1, dinvb, b1r, Wf)

    zp2 = _sc_propagate(g2, src2d, dst2d)
    return _tc_final(zp2, g2, dinvb, bfr)


# speedup vs baseline: 1.0090x; 1.0090x over previous
import dataclasses
import functools

import jax
import jax.numpy as jnp
from jax import lax
from jax.experimental import pallas as pl
from jax.experimental.pallas import tpu as pltpu
from jax.experimental.pallas import tpu_sc as plsc

N = 10000
E = 320000
D = 128

NC = 2
NS = 16
NW = NC * NS
CHUNK = 125
NCHUNK = E // CHUNK
CPW = NCHUNK // NW
IQ = 16
RMAIN = 624
RTAIL = N - RMAIN * NS

_mesh = plsc.VectorSubcoreMesh(core_axis_name="c", subcore_axis_name="s")

_cp_no_layout = pltpu.CompilerParams()
if "needs_layout_passes" in pltpu.CompilerParams.__dataclass_fields__:
    _cp_no_layout = dataclasses.replace(_cp_no_layout,
                                        needs_layout_passes=False)


def _each_row_slice(s, fn):
    fn(s * RMAIN, RMAIN)

    @pl.when(s < RTAIL // 8)
    def _():
        fn(RMAIN * NS + s * 8, 8)


DROWS = 80
EPW = E // NW
NVEC = EPW // 16


def _sc_degree(dst1d):
    @functools.partial(
        pl.kernel,
        mesh=_mesh,
        out_type=jax.ShapeDtypeStruct((NC, DROWS, D), jnp.float32),
        compiler_params=_cp_no_layout,
        scratch_types=[
            pltpu.VMEM((EPW,), jnp.int32),
            pltpu.VMEM((8, DROWS, D), jnp.float32),
            pltpu.VMEM((1, DROWS), jnp.int32),
            pltpu.VMEM_SHARED((DROWS, D), jnp.float32),
        ],
    )
    def k(dst_hbm, out_hbm, idx_v, hist, rowid, accd):
        c = lax.axis_index("c")
        s = lax.axis_index("s")
        w = s * NC + c
        pltpu.sync_copy(dst_hbm.at[pl.ds(w * EPW, EPW)], idx_v)

        zeros16 = jnp.zeros((16,), jnp.float32)
        iota = lax.iota(jnp.int32, 16)

        @pl.loop(0, 8)
        def _(p):
            @pl.loop(0, DROWS)
            def _(r):
                @pl.loop(0, D // 16)
                def _(cc):
                    hist[p, r, pl.ds(cc * 16, 16)] = zeros16

        @pl.loop(0, DROWS // 16)
        def _(kk):
            rowid[0, pl.ds(kk * 16, 16)] = iota + kk * 16

        @pl.when(s == 0)
        def _():
            pltpu.sync_copy(hist.at[0], accd)

        plane = iota & 7
        mask_lo = iota < 8
        mask_hi = iota >= 8
        ones16 = jnp.ones((16,), jnp.float32)

        @pl.loop(0, NVEC)
        def _(v):
            d = idx_v[pl.ds(v * 16, 16)]
            r = lax.shift_right_logical(d, 7)
            col = d & 127
            plsc.addupdate_scatter(hist, [plane, r, col], ones16, mask=mask_lo)
            plsc.addupdate_scatter(hist, [plane, r, col], ones16, mask=mask_hi)

        plsc.subcore_barrier()

        @pl.loop(0, 8)
        def _(p):
            pltpu.sync_copy(hist.at[p], accd.at[rowid.at[0]], add=True)

        plsc.subcore_barrier()

        @pl.when(s < DROWS // 8)
        def _():
            pltpu.sync_copy(accd.at[pl.ds(s * 8, 8)],
                            out_hbm.at[c, pl.ds(s * 8, 8)])

    return k(dst1d)


def _sc_propagate(g, src2d, dst2d):
    @functools.partial(
        pl.kernel,
        mesh=_mesh,
        out_type=jax.ShapeDtypeStruct((NC, N, D), jnp.float32),
        scratch_types=[
            pltpu.VMEM((IQ, CHUNK), jnp.int32),
            pltpu.VMEM((IQ, CHUNK), jnp.int32),
            pltpu.VMEM((2, CHUNK, D), jnp.float32),
            pltpu.VMEM_SHARED((N, D), jnp.float32),
            pltpu.SemaphoreType.DMA((2,)),
            pltpu.SemaphoreType.DMA((2,)),
        ],
    )
    def k(g_hbm, src_hbm, dst_hbm, out_hbm, sidx, didx, rows, acc, gsem, ssem):
        c = lax.axis_index("c")
        s = lax.axis_index("s")
        w = s * NC + c
        pltpu.sync_copy(src_hbm.at[pl.ds(w * CPW, IQ)], sidx)
        pltpu.sync_copy(dst_hbm.at[pl.ds(w * CPW, IQ)], didx)
        _each_row_slice(s, lambda st, sz: pltpu.sync_copy(
            g_hbm.at[pl.ds(st, sz)], acc.at[pl.ds(st, sz)]))
        plsc.subcore_barrier()

        def g_start(j):
            pltpu.async_copy(g_hbm.at[sidx.at[j % IQ]], rows.at[j % 2],
                             gsem.at[j % 2])

        def g_wait(j):
            pltpu.make_async_copy(g_hbm.at[sidx.at[j % IQ]], rows.at[j % 2],
                                  gsem.at[j % 2]).wait()

        def s_start(j):
            pltpu.async_copy(rows.at[j % 2], acc.at[didx.at[j % IQ]],
                             ssem.at[j % 2], add=True)

        def s_wait(j):
            pltpu.make_async_copy(rows.at[j % 2], acc.at[didx.at[j % IQ]],
                                  ssem.at[j % 2]).wait()

        g_start(0)

        @pl.loop(0, CPW)
        def _(j):
            g_wait(j)

            @pl.when(j >= 1)
            def _():
                s_wait(j - 1)

            @pl.when(j < CPW - 1)
            def _():
                jj = j + 1
                @pl.when(jj % IQ == 0)
                def _():
                    off = pl.multiple_of(w * CPW + jj, 8)
                    pltpu.sync_copy(src_hbm.at[pl.ds(off, IQ)], sidx)

                g_start(jj)

            @pl.when(jnp.logical_and(j % IQ == 0, j > 0))
            def _():
                off = pl.multiple_of(w * CPW + j, 8)
                pltpu.sync_copy(dst_hbm.at[pl.ds(off, IQ)], didx)

            s_start(j)

        s_wait(CPW - 1)
        plsc.subcore_barrier()
        _each_row_slice(s, lambda st, sz: pltpu.sync_copy(
            acc.at[pl.ds(st, sz)], out_hbm.at[c, pl.ds(st, sz)]))

    return k(g, src2d, dst2d)


def _tc_mm_scale0(x, W0, degp2):
    def body(x_ref, w_ref, degp_ref, g_ref, dinvb_ref):
        deg = degp_ref[0] + degp_ref[1] + 1.0
        dinvb = jnp.broadcast_to(lax.rsqrt(deg), (N, D))
        dinvb_ref[...] = dinvb
        h0 = jnp.dot(x_ref[...], w_ref[...],
                     preferred_element_type=jnp.float32)
        g_ref[...] = dinvb * h0

    return pl.pallas_call(
        body,
        out_shape=(jax.ShapeDtypeStruct((N, D), jnp.float32),
                   jax.ShapeDtypeStruct((N, D), jnp.float32)))(x, W0, degp2)


def _tc_bn_relu_mm(zp, g, dinvb, b0, gamma0, beta0, W1):
    def body(zp_ref, g_ref, dinvb_ref, b_ref, gam_ref, bet_ref, w_ref, o_ref):
        dinvb = dinvb_ref[...]
        z = zp_ref[0] + zp_ref[1] - g_ref[...]
        y = dinvb * z + b_ref[...]
        mean = jnp.mean(y, axis=0, keepdims=True)
        var = jnp.mean((y - mean) ** 2, axis=0, keepdims=True)
        xn = (y - mean) * lax.rsqrt(var + 1e-5) * gam_ref[...] + bet_ref[...]
        xr = jnp.maximum(xn, 0.0)
        h1 = jnp.dot(xr, w_ref[...], preferred_element_type=jnp.float32)
        o_ref[...] = dinvb * h1

    return pl.pallas_call(
        body, out_shape=jax.ShapeDtypeStruct((N, D), jnp.float32))(
            zp, g, dinvb, b0, gamma0, beta0, W1)


def _tc_combine_mm(zp, g, dinvb, b1, Wf):
    def body(zp_ref, g_ref, dinvb_ref, b_ref, w_ref, o_ref):
        dinvb = dinvb_ref[...]
        z = zp_ref[0] + zp_ref[1] - g_ref[...]
        y = dinvb * z + b_ref[...]
        h2 = jnp.dot(y, w_ref[...], preferred_element_type=jnp.float32)
        o_ref[...] = dinvb * h2

    return pl.pallas_call(
        body, out_shape=jax.ShapeDtypeStruct((N, D), jnp.float32))(
            zp, g, dinvb, b1, Wf)


def _tc_final(zp, g, dinvb, bf):
    def body(zp_ref, g_ref, dinvb_ref, b_ref, o_ref):
        z = zp_ref[0] + zp_ref[1] - g_ref[...]
        o = dinvb_ref[...] * z + b_ref[...]
        m = jnp.max(o, axis=1, keepdims=True)
        lse = jnp.log(jnp.sum(jnp.exp(o - m), axis=1, keepdims=True)) + m
        o_ref[...] = o - lse

    return pl.pallas_call(
        body, out_shape=jax.ShapeDtypeStruct((N, D), jnp.float32))(
            zp, g, dinvb, bf)


def kernel(x, edge_index, W0, b0, gamma0, beta0, W1, b1, Wf, bf):
    ei = edge_index.astype(jnp.int32)
    src2d = ei[0].reshape(NCHUNK, CHUNK)
    dst2d = ei[1].reshape(NCHUNK, CHUNK)
    b0r = b0.reshape(1, D)
    gam = gamma0.reshape(1, D)
    bet = beta0.reshape(1, D)
    b1r = b1.reshape(1, D)
    bfr = bf.reshape(1, D)

    degp = _sc_degree(ei[1])
    degp2 = degp.reshape(NC, DROWS * D)[:, :N, None]
    g0, dinvb = _tc_mm_scale0(x, W0, degp2)

    zp0 = _sc_propagate(g0, src2d, dst2d)
    g1 = _tc_bn_relu_mm(zp0, g0, dinvb, b0r, gam, bet, W1)

    zp1 = _sc_propagate(g1, src2d, dst2d)
    g2 = _tc_combine_mm(zp1, g1, dinvb, b1r, Wf)

    zp2 = _sc_propagate(g2, src2d, dst2d)
    return _tc_final(zp2, g2, dinvb, bfr)
